# baseline (speedup 1.0000x reference)
import functools

import jax
import jax.numpy as jnp
from jax import lax
from jax.experimental import pallas as pl
from jax.experimental.pallas import tpu as pltpu
from jax.experimental.pallas import tpu_sc as plsc

_N = 10000
_E = 320000
_F = 128
_H = 128
_C = 40
_EPS = 1e-7


_R = 2000
_G = _N // _R


def _table(h2, t_ref):
    u = h2 + _EPS
    q = jnp.exp(u)
    t_ref[0] = u * q
    t_ref[1] = q


def _enc_body(x_ref, w_ref, b_ref, h_ref, t_ref):
    h = jnp.dot(x_ref[...], w_ref[...], preferred_element_type=jnp.float32)
    h = h + b_ref[...]
    h_ref[...] = h
    _table(jnp.maximum(h, 0.0), t_ref)


def _ln_relu(hc, g_ref, be_ref):
    mu = jnp.mean(hc, axis=-1, keepdims=True)
    d = hc - mu
    var = jnp.mean(d * d, axis=-1, keepdims=True)
    h1 = d / jnp.sqrt(var + 1e-5) * g_ref[...] + be_ref[...]
    return jnp.maximum(h1, 0.0)


def _combine(hin_ref, res_ref, s_ref, w_ref, b_ref):
    m = s_ref[0] / (s_ref[1] + 1e-16)
    hc = jnp.dot(hin_ref[...] + m, w_ref[...],
                 preferred_element_type=jnp.float32)
    return hc + b_ref[...] + res_ref[...]


def _mid_body(hin_ref, res_ref, s_ref, w_ref, b_ref, g_ref, be_ref,
              hc_ref, h2_ref, t_ref):
    hc = _combine(hin_ref, res_ref, s_ref, w_ref, b_ref)
    hc_ref[...] = hc
    h2 = _ln_relu(hc, g_ref, be_ref)
    h2_ref[...] = h2
    _table(h2, t_ref)


def _fin_body(hin_ref, res_ref, s_ref, w_ref, b_ref, g_ref, be_ref,
              wp_ref, bp_ref, o_ref):
    hc = _combine(hin_ref, res_ref, s_ref, w_ref, b_ref)
    h3 = _ln_relu(hc, g_ref, be_ref)
    z = jnp.dot(h3, wp_ref[...], preferred_element_type=jnp.float32)
    z = z + bp_ref[...]
    zm = jnp.max(z, axis=-1, keepdims=True)
    lse = jnp.log(jnp.sum(jnp.exp(z - zm), axis=-1, keepdims=True))
    o_ref[...] = z - zm - lse


_rowspec = pl.BlockSpec((_R, _H), lambda i: (i, 0))
_sspec = pl.BlockSpec((2, _R, _H), lambda i: (0, i, 0))
_wspec = pl.BlockSpec((_H, _H), lambda i: (0, 0))
_vspec = pl.BlockSpec((1, _H), lambda i: (0, 0))

_enc = pl.pallas_call(
    _enc_body,
    grid=(_G,),
    in_specs=[pl.BlockSpec((_R, _F), lambda i: (i, 0)), _wspec, _vspec],
    out_specs=[_rowspec, _sspec],
    out_shape=[jax.ShapeDtypeStruct((_N, _H), jnp.float32),
               jax.ShapeDtypeStruct((2, _N, _H), jnp.float32)],
)

_mid = pl.pallas_call(
    _mid_body,
    grid=(_G,),
    in_specs=[_rowspec, _rowspec, _sspec, _wspec, _vspec, _vspec, _vspec],
    out_specs=[_rowspec, _rowspec, _sspec],
    out_shape=[jax.ShapeDtypeStruct((_N, _H), jnp.float32),
               jax.ShapeDtypeStruct((_N, _H), jnp.float32),
               jax.ShapeDtypeStruct((2, _N, _H), jnp.float32)],
)

_fin = pl.pallas_call(
    _fin_body,
    grid=(_G,),
    in_specs=[_rowspec, _rowspec, _sspec, _wspec, _vspec, _vspec, _vspec,
              pl.BlockSpec((_H, _C), lambda i: (0, 0)),
              pl.BlockSpec((1, _C), lambda i: (0, 0))],
    out_specs=pl.BlockSpec((_R, _C), lambda i: (i, 0)),
    out_shape=jax.ShapeDtypeStruct((_N, _C), jnp.float32),
)


_NSUB = 16
_CH = 80
_GC = 63
_NG = 4
_EPT = _CH * _GC * _NG
_EPAD = _NSUB * _EPT
_NPAD = _N + 8
_RPT = 624
_RTAIL = _N - _NSUB * _RPT


def _agg_body(t_hbm, src2_hbm, dst_hbm, zro_hbm, out_hbm,
              sidx, didx, rows, g0, g1, g2, s0, s1, s2, acc):
    c = lax.axis_index("c")
    s = lax.axis_index("s")
    r0 = pl.multiple_of(s * _RPT, 8)
    pltpu.sync_copy(zro_hbm.at[pl.ds(r0, _RPT)], acc.at[pl.ds(r0, _RPT)])

    @pl.when(s == _NSUB - 1)
    def _():
        pltpu.sync_copy(zro_hbm.at[pl.ds(_NSUB * _RPT, _RTAIL)],
                        acc.at[pl.ds(_NSUB * _RPT, _RTAIL)])

    gsem = (g0, g1, g2)
    ssem = (s0, s1, s2)

    def start_gather(k, b):
        pltpu.async_copy(t_hbm.at[sidx.at[k]], rows.at[b], gsem[b])

    def wait_gather(k, b):
        pltpu.make_async_copy(t_hbm.at[sidx.at[k]], rows.at[b],
                              gsem[b]).wait()

    def start_scatter(k, b):
        pltpu.async_copy(rows.at[b], acc.at[didx.at[k]], ssem[b], add=True)

    def wait_scatter(k, b):
        pltpu.make_async_copy(rows.at[b], acc.at[didx.at[k]],
                              ssem[b]).wait()

    plsc.subcore_barrier()

    def group(g, carry):
        pltpu.sync_copy(src2_hbm.at[c, s, g], sidx)
        pltpu.sync_copy(dst_hbm.at[s, g], didx)
        start_gather(0, 0)
        start_gather(1, 1)
        wait_gather(0, 0)
        start_scatter(0, 0)
        start_gather(2, 2)

        def step(i, carry2):
            for u in range(3):
                k = 3 * i + 1 + u
                b = (1 + u) % 3
                wait_gather(k, b)
                start_scatter(k, b)
                wait_scatter(k - 1, u)
                start_gather(k + 2, u)
            return carry2

        lax.fori_loop(0, (_GC - 3) // 3, step, 0)
        for k in (_GC - 2, _GC - 1):
            b = k % 3
            wait_gather(k, b)
            start_scatter(k, b)
            wait_scatter(k - 1, (k - 1) % 3)
        wait_scatter(_GC - 1, (_GC - 1) % 3)
        return carry

    lax.fori_loop(0, _NG, group, 0)
    plsc.subcore_barrier()
    pltpu.sync_copy(acc.at[pl.ds(r0, _RPT)],
                    out_hbm.at[pl.ds(pl.multiple_of(c * _N + r0, 8), _RPT)])

    @pl.when(s == _NSUB - 1)
    def _():
        o0 = pl.multiple_of(c * _N + _NSUB * _RPT, 8)
        pltpu.sync_copy(acc.at[pl.ds(_NSUB * _RPT, _RTAIL)],
                        out_hbm.at[pl.ds(o0, _RTAIL)])


@functools.cache
def _agg():
    return pl.kernel(
        _agg_body,
        out_type=jax.ShapeDtypeStruct((2 * _N, _H), jnp.float32),
        mesh=plsc.VectorSubcoreMesh(core_axis_name="c", subcore_axis_name="s"),
        scratch_types=[
            pltpu.VMEM((_GC, _CH), jnp.int32),
            pltpu.VMEM((_GC, _CH), jnp.int32),
            pltpu.VMEM((3, _CH, _H), jnp.float32),
            pltpu.SemaphoreType.DMA,
            pltpu.SemaphoreType.DMA,
            pltpu.SemaphoreType.DMA,
            pltpu.SemaphoreType.DMA,
            pltpu.SemaphoreType.DMA,
            pltpu.SemaphoreType.DMA,
            pltpu.VMEM_SHARED((_NPAD, _H), jnp.float32),
        ],
    )


def kernel(x, edge_index, W_enc, b_enc, W0, b0, g0, be0, W1, b1, g1, be1,
           W2, b2, g2, be2, W_pred, b_pred):
    npad = _EPAD - _E
    srcp = jnp.concatenate([edge_index[0],
                            jnp.zeros((npad,), jnp.int32)])
    dstp = jnp.concatenate([edge_index[1],
                            jnp.full((npad,), _N, jnp.int32)])
    src2 = jnp.stack([srcp, srcp + _N]).reshape(2, _NSUB, _NG, _GC, _CH)
    dstc = dstp.reshape(_NSUB, _NG, _GC, _CH)
    zro = jnp.zeros((_N, _H), jnp.float32)
    r = lambda v: v.reshape(1, -1)

    agg = _agg()
    h, t0 = _enc(x, W_enc, r(b_enc))
    s0 = agg(t0.reshape(2 * _N, _H), src2, dstc, zro).reshape(2, _N, _H)
    ha, h2a, t1 = _mid(h, zro, s0, W0, r(b0), r(g0), r(be0))
    s1 = agg(t1.reshape(2 * _N, _H), src2, dstc, zro).reshape(2, _N, _H)
    hb, h2b, t2 = _mid(h2a, ha, s1, W1, r(b1), r(g1), r(be1))
    s2 = agg(t2.reshape(2 * _N, _H), src2, dstc, zro).reshape(2, _N, _H)
    return _fin(h2b, hb, s2, W2, r(b2), r(g2), r(be2), W_pred, r(b_pred))


# speedup vs baseline: 1.4133x; 1.4133x over previous
import functools

import jax
import jax.numpy as jnp
from jax import lax
from jax.experimental import pallas as pl
from jax.experimental.pallas import tpu as pltpu
from jax.experimental.pallas import tpu_sc as plsc

_N = 10000
_E = 320000
_F = 128
_H = 128
_C = 40
_EPS = 1e-7


_R = 2000
_G = _N // _R


def _table(h2, t_ref):
    u = h2 + _EPS
    q = jnp.exp(u)
    t_ref[0] = u * q
    t_ref[1] = q


def _enc_body(x_ref, w_ref, b_ref, h_ref, t_ref):
    h = jnp.dot(x_ref[...], w_ref[...], preferred_element_type=jnp.float32)
    h = h + b_ref[...]
    h_ref[...] = h
    _table(jnp.maximum(h, 0.0), t_ref)


def _ln_relu(hc, g_ref, be_ref):
    mu = jnp.mean(hc, axis=-1, keepdims=True)
    d = hc - mu
    var = jnp.mean(d * d, axis=-1, keepdims=True)
    h1 = d / jnp.sqrt(var + 1e-5) * g_ref[...] + be_ref[...]
    return jnp.maximum(h1, 0.0)


def _combine(hin_ref, res_ref, s_ref, w_ref, b_ref):
    m = s_ref[0] / (s_ref[1] + 1e-16)
    hc = jnp.dot(hin_ref[...] + m, w_ref[...],
                 preferred_element_type=jnp.float32)
    return hc + b_ref[...] + res_ref[...]


def _mid_body(hin_ref, res_ref, s_ref, w_ref, b_ref, g_ref, be_ref,
              hc_ref, h2_ref, t_ref):
    hc = _combine(hin_ref, res_ref, s_ref, w_ref, b_ref)
    hc_ref[...] = hc
    h2 = _ln_relu(hc, g_ref, be_ref)
    h2_ref[...] = h2
    _table(h2, t_ref)


def _fin_body(hin_ref, res_ref, s_ref, w_ref, b_ref, g_ref, be_ref,
              wp_ref, bp_ref, o_ref):
    hc = _combine(hin_ref, res_ref, s_ref, w_ref, b_ref)
    h3 = _ln_relu(hc, g_ref, be_ref)
    z = jnp.dot(h3, wp_ref[...], preferred_element_type=jnp.float32)
    z = z + bp_ref[...]
    zm = jnp.max(z, axis=-1, keepdims=True)
    lse = jnp.log(jnp.sum(jnp.exp(z - zm), axis=-1, keepdims=True))
    o_ref[...] = z - zm - lse


_rowspec = pl.BlockSpec((_R, _H), lambda i: (i, 0))
_sspec = pl.BlockSpec((2, _R, _H), lambda i: (0, i, 0))
_wspec = pl.BlockSpec((_H, _H), lambda i: (0, 0))
_vspec = pl.BlockSpec((1, _H), lambda i: (0, 0))

_enc = pl.pallas_call(
    _enc_body,
    grid=(_G,),
    in_specs=[pl.BlockSpec((_R, _F), lambda i: (i, 0)), _wspec, _vspec],
    out_specs=[_rowspec, _sspec],
    out_shape=[jax.ShapeDtypeStruct((_N, _H), jnp.float32),
               jax.ShapeDtypeStruct((2, _N, _H), jnp.float32)],
)

_mid = pl.pallas_call(
    _mid_body,
    grid=(_G,),
    in_specs=[_rowspec, _rowspec, _sspec, _wspec, _vspec, _vspec, _vspec],
    out_specs=[_rowspec, _rowspec, _sspec],
    out_shape=[jax.ShapeDtypeStruct((_N, _H), jnp.float32),
               jax.ShapeDtypeStruct((_N, _H), jnp.float32),
               jax.ShapeDtypeStruct((2, _N, _H), jnp.float32)],
)

_fin = pl.pallas_call(
    _fin_body,
    grid=(_G,),
    in_specs=[_rowspec, _rowspec, _sspec, _wspec, _vspec, _vspec, _vspec,
              pl.BlockSpec((_H, _C), lambda i: (0, 0)),
              pl.BlockSpec((1, _C), lambda i: (0, 0))],
    out_specs=pl.BlockSpec((_R, _C), lambda i: (i, 0)),
    out_shape=jax.ShapeDtypeStruct((_N, _C), jnp.float32),
)


_NSUB = 16
_EPT = _E // _NSUB
_CH = 125
_GC = 32
_NG = _EPT // (_CH * _GC)
_RPT = 624
_RTAIL = _N - _NSUB * _RPT


def _agg_body(t_hbm, src2_hbm, dst_hbm, zro_hbm, out_hbm,
              sidx, didx, rows, sem0, sem1, acc):
    c = lax.axis_index("c")
    s = lax.axis_index("s")
    r0 = pl.multiple_of(s * _RPT, 8)
    pltpu.sync_copy(zro_hbm.at[pl.ds(r0, _RPT)], acc.at[pl.ds(r0, _RPT)])

    @pl.when(s == _NSUB - 1)
    def _():
        pltpu.sync_copy(zro_hbm.at[pl.ds(_NSUB * _RPT, _RTAIL)],
                        acc.at[pl.ds(_NSUB * _RPT, _RTAIL)])

    sems = (sem0, sem1)

    def start_gather(k, b):
        pltpu.async_copy(t_hbm.at[sidx.at[k]], rows.at[b], sems[b])

    def wait_gather(k, b):
        pltpu.make_async_copy(t_hbm.at[sidx.at[k]], rows.at[b],
                              sems[b]).wait()

    plsc.subcore_barrier()

    def group(g, carry):
        pltpu.sync_copy(src2_hbm.at[c, s, g], sidx)
        pltpu.sync_copy(dst_hbm.at[s, g], didx)
        start_gather(0, 0)
        start_gather(1, 1)

        def step(i, carry2):
            k0 = 2 * i
            for b in range(2):
                k = k0 + b
                wait_gather(k, b)
                pltpu.sync_copy(rows.at[b], acc.at[didx.at[k]], add=True)
                start_gather(k + 2, b)
            return carry2

        lax.fori_loop(0, _GC // 2 - 1, step, 0)
        for b in range(2):
            k = _GC - 2 + b
            wait_gather(k, b)
            pltpu.sync_copy(rows.at[b], acc.at[didx.at[k]], add=True)
        return carry

    lax.fori_loop(0, _NG, group, 0)
    plsc.subcore_barrier()
    pltpu.sync_copy(acc.at[pl.ds(r0, _RPT)],
                    out_hbm.at[pl.ds(pl.multiple_of(c * _N + r0, 8), _RPT)])

    @pl.when(s == _NSUB - 1)
    def _():
        o0 = pl.multiple_of(c * _N + _NSUB * _RPT, 8)
        pltpu.sync_copy(acc.at[pl.ds(_NSUB * _RPT, _RTAIL)],
                        out_hbm.at[pl.ds(o0, _RTAIL)])


@functools.cache
def _agg():
    return pl.kernel(
        _agg_body,
        out_type=jax.ShapeDtypeStruct((2 * _N, _H), jnp.float32),
        mesh=plsc.VectorSubcoreMesh(core_axis_name="c", subcore_axis_name="s"),
        scratch_types=[
            pltpu.VMEM((_GC, _CH), jnp.int32),
            pltpu.VMEM((_GC, _CH), jnp.int32),
            pltpu.VMEM((2, _CH, _H), jnp.float32),
            pltpu.SemaphoreType.DMA,
            pltpu.SemaphoreType.DMA,
            pltpu.VMEM_SHARED((_N, _H), jnp.float32),
        ],
    )


def kernel(x, edge_index, W_enc, b_enc, W0, b0, g0, be0, W1, b1, g1, be1,
           W2, b2, g2, be2, W_pred, b_pred):
    src = edge_index[0]
    dst = edge_index[1]
    src2 = jnp.stack([src, src + _N]).reshape(2, _NSUB, _NG, _GC, _CH)
    dstc = dst.reshape(_NSUB, _NG, _GC, _CH)
    zro = jnp.zeros((_N, _H), jnp.float32)
    r = lambda v: v.reshape(1, -1)

    agg = _agg()
    h, t0 = _enc(x, W_enc, r(b_enc))
    s0 = agg(t0.reshape(2 * _N, _H), src2, dstc, zro).reshape(2, _N, _H)
    ha, h2a, t1 = _mid(h, zro, s0, W0, r(b0), r(g0), r(be0))
    s1 = agg(t1.reshape(2 * _N, _H), src2, dstc, zro).reshape(2, _N, _H)
    hb, h2b, t2 = _mid(h2a, ha, s1, W1, r(b1), r(g1), r(be1))
    s2 = agg(t2.reshape(2 * _N, _H), src2, dstc, zro).reshape(2, _N, _H)
    return _fin(h2b, hb, s2, W2, r(b2), r(g2), r(be2), W_pred, r(b_pred))


# speedup vs baseline: 1.4221x; 1.0062x over previous
import functools

import jax
import jax.numpy as jnp
from jax import lax
from jax.experimental import pallas as pl
from jax.experimental.pallas import tpu as pltpu
from jax.experimental.pallas import tpu_sc as plsc

_N = 10000
_E = 320000
_F = 128
_H = 128
_C = 40
_EPS = 1e-7


_R = 2000
_G = _N // _R


def _table(h2, t_ref):
    u = h2 + _EPS
    q = jnp.exp(u)
    t_ref[0] = u * q
    t_ref[1] = q


def _enc_body(x_ref, w_ref, b_ref, h_ref, t_ref):
    h = jnp.dot(x_ref[...], w_ref[...], preferred_element_type=jnp.float32)
    h = h + b_ref[...]
    h_ref[...] = h
    _table(jnp.maximum(h, 0.0), t_ref)


def _ln_relu(hc, g_ref, be_ref):
    mu = jnp.mean(hc, axis=-1, keepdims=True)
    d = hc - mu
    var = jnp.mean(d * d, axis=-1, keepdims=True)
    h1 = d / jnp.sqrt(var + 1e-5) * g_ref[...] + be_ref[...]
    return jnp.maximum(h1, 0.0)


def _combine(hin_ref, res_ref, s_ref, w_ref, b_ref):
    m = s_ref[0] / (s_ref[1] + 1e-16)
    hc = jnp.dot(hin_ref[...] + m, w_ref[...],
                 preferred_element_type=jnp.float32)
    return hc + b_ref[...] + res_ref[...]


def _mid_body(hin_ref, res_ref, s_ref, w_ref, b_ref, g_ref, be_ref,
              hc_ref, h2_ref, t_ref):
    hc = _combine(hin_ref, res_ref, s_ref, w_ref, b_ref)
    hc_ref[...] = hc
    h2 = _ln_relu(hc, g_ref, be_ref)
    h2_ref[...] = h2
    _table(h2, t_ref)


def _fin_body(hin_ref, res_ref, s_ref, w_ref, b_ref, g_ref, be_ref,
              wp_ref, bp_ref, o_ref):
    hc = _combine(hin_ref, res_ref, s_ref, w_ref, b_ref)
    h3 = _ln_relu(hc, g_ref, be_ref)
    z = jnp.dot(h3, wp_ref[...], preferred_element_type=jnp.float32)
    z = z + bp_ref[...]
    zm = jnp.max(z, axis=-1, keepdims=True)
    lse = jnp.log(jnp.sum(jnp.exp(z - zm), axis=-1, keepdims=True))
    o_ref[...] = z - zm - lse


_rowspec = pl.BlockSpec((_R, _H), lambda i: (i, 0))
_sspec = pl.BlockSpec((2, _R, _H), lambda i: (0, i, 0))
_wspec = pl.BlockSpec((_H, _H), lambda i: (0, 0))
_vspec = pl.BlockSpec((1, _H), lambda i: (0, 0))

_enc = pl.pallas_call(
    _enc_body,
    grid=(_G,),
    in_specs=[pl.BlockSpec((_R, _F), lambda i: (i, 0)), _wspec, _vspec],
    out_specs=[_rowspec, _sspec],
    out_shape=[jax.ShapeDtypeStruct((_N, _H), jnp.float32),
               jax.ShapeDtypeStruct((2, _N, _H), jnp.float32)],
)

_mid = pl.pallas_call(
    _mid_body,
    grid=(_G,),
    in_specs=[_rowspec, _rowspec, _sspec, _wspec, _vspec, _vspec, _vspec],
    out_specs=[_rowspec, _rowspec, _sspec],
    out_shape=[jax.ShapeDtypeStruct((_N, _H), jnp.float32),
               jax.ShapeDtypeStruct((_N, _H), jnp.float32),
               jax.ShapeDtypeStruct((2, _N, _H), jnp.float32)],
)

_fin = pl.pallas_call(
    _fin_body,
    grid=(_G,),
    in_specs=[_rowspec, _rowspec, _sspec, _wspec, _vspec, _vspec, _vspec,
              pl.BlockSpec((_H, _C), lambda i: (0, 0)),
              pl.BlockSpec((1, _C), lambda i: (0, 0))],
    out_specs=pl.BlockSpec((_R, _C), lambda i: (i, 0)),
    out_shape=jax.ShapeDtypeStruct((_N, _C), jnp.float32),
)


_NSUB = 16
_EPT = _E // _NSUB
_CH = 125
_GC = 40
_NG = _EPT // (_CH * _GC)
_RPT = 624
_RTAIL = _N - _NSUB * _RPT


def _agg_body(t_hbm, src2_hbm, dst_hbm, zro_hbm, out_hbm,
              sidx, didx, rows, sem0, sem1, acc):
    c = lax.axis_index("c")
    s = lax.axis_index("s")
    r0 = pl.multiple_of(s * _RPT, 8)
    pltpu.sync_copy(zro_hbm.at[pl.ds(r0, _RPT)], acc.at[pl.ds(r0, _RPT)])

    @pl.when(s == _NSUB - 1)
    def _():
        pltpu.sync_copy(zro_hbm.at[pl.ds(_NSUB * _RPT, _RTAIL)],
                        acc.at[pl.ds(_NSUB * _RPT, _RTAIL)])

    sems = (sem0, sem1)

    def start_gather(k, b):
        pltpu.async_copy(t_hbm.at[sidx.at[k]], rows.at[b], sems[b])

    def wait_gather(k, b):
        pltpu.make_async_copy(t_hbm.at[sidx.at[k]], rows.at[b],
                              sems[b]).wait()

    plsc.subcore_barrier()

    def group(g, carry):
        pltpu.sync_copy(src2_hbm.at[c, s, g], sidx)
        pltpu.sync_copy(dst_hbm.at[s, g], didx)
        start_gather(0, 0)
        start_gather(1, 1)

        def step(i, carry2):
            k0 = 2 * i
            for b in range(2):
                k = k0 + b
                wait_gather(k, b)
                pltpu.sync_copy(rows.at[b], acc.at[didx.at[k]], add=True)
                start_gather(k + 2, b)
            return carry2

        lax.fori_loop(0, _GC // 2 - 1, step, 0)
        for b in range(2):
            k = _GC - 2 + b
            wait_gather(k, b)
            pltpu.sync_copy(rows.at[b], acc.at[didx.at[k]], add=True)
        return carry

    lax.fori_loop(0, _NG, group, 0)
    plsc.subcore_barrier()
    pltpu.sync_copy(acc.at[pl.ds(r0, _RPT)],
                    out_hbm.at[pl.ds(pl.multiple_of(c * _N + r0, 8), _RPT)])

    @pl.when(s == _NSUB - 1)
    def _():
        o0 = pl.multiple_of(c * _N + _NSUB * _RPT, 8)
        pltpu.sync_copy(acc.at[pl.ds(_NSUB * _RPT, _RTAIL)],
                        out_hbm.at[pl.ds(o0, _RTAIL)])


@functools.cache
def _agg():
    return pl.kernel(
        _agg_body,
        out_type=jax.ShapeDtypeStruct((2 * _N, _H), jnp.float32),
        mesh=plsc.VectorSubcoreMesh(core_axis_name="c", subcore_axis_name="s"),
        scratch_types=[
            pltpu.VMEM((_GC, _CH), jnp.int32),
            pltpu.VMEM((_GC, _CH), jnp.int32),
            pltpu.VMEM((2, _CH, _H), jnp.float32),
            pltpu.SemaphoreType.DMA,
            pltpu.SemaphoreType.DMA,
            pltpu.VMEM_SHARED((_N, _H), jnp.float32),
        ],
    )


def kernel(x, edge_index, W_enc, b_enc, W0, b0, g0, be0, W1, b1, g1, be1,
           W2, b2, g2, be2, W_pred, b_pred):
    src = edge_index[0]
    dst = edge_index[1]
    src2 = jnp.stack([src, src + _N]).reshape(2, _NSUB, _NG, _GC, _CH)
    dstc = dst.reshape(_NSUB, _NG, _GC, _CH)
    zro = jnp.zeros((_N, _H), jnp.float32)
    r = lambda v: v.reshape(1, -1)

    agg = _agg()
    h, t0 = _enc(x, W_enc, r(b_enc))
    s0 = agg(t0.reshape(2 * _N, _H), src2, dstc, zro).reshape(2, _N, _H)
    ha, h2a, t1 = _mid(h, zro, s0, W0, r(b0), r(g0), r(be0))
    s1 = agg(t1.reshape(2 * _N, _H), src2, dstc, zro).reshape(2, _N, _H)
    hb, h2b, t2 = _mid(h2a, ha, s1, W1, r(b1), r(g1), r(be1))
    s2 = agg(t2.reshape(2 * _N, _H), src2, dstc, zro).reshape(2, _N, _H)
    return _fin(h2b, hb, s2, W2, r(b2), r(g2), r(be2), W_pred, r(b_pred))
